# baseline (speedup 1.0000x reference)
import functools

import jax
import jax.numpy as jnp
from jax import lax
from jax.experimental import pallas as pl
from jax.experimental.pallas import tpu as pltpu
from jax.experimental.pallas import tpu_sc as plsc

B_, S_, D_ = 2, 2048, 1024
H_ = 4096
E_ = 8
K_ = 2
N_ = B_ * S_
CAP = int(round(K_ * N_ * 1.25 / E_))
NSLOT = E_ * CAP
SZ = NSLOT + 32
NOISE = 1.0 / E_
NC, NS = 2, 16
NW = NC * NS
ROWS_W = NSLOT // NW
TOK_W = N_ // NW
ASG_W = (N_ * K_) // NW


def _sc_mesh():
    return plsc.VectorSubcoreMesh(
        core_axis_name="c", subcore_axis_name="s",
        num_cores=NC, num_subcores=NS)


def _wid():
    return lax.axis_index("s") * NC + lax.axis_index("c")


def _router_body(lg_ref, scs_ref, cbs_ref, w0_ref, w1_ref, sums_ref):
    lg = [lg_ref[e] for e in range(E_)]
    neg = jnp.float32(-jnp.inf)

    m0 = lg[0]
    for e in range(1, E_):
        m0 = jnp.maximum(m0, lg[e])
    e0 = jnp.full((32, 128), E_, jnp.int32)
    for e in range(E_ - 1, -1, -1):
        e0 = jnp.where(lg[e] == m0, e, e0)

    m1 = jnp.full((32, 128), neg)
    for e in range(E_):
        le = jnp.where(e0 == e, neg, lg[e])
        m1 = jnp.maximum(m1, le)
    e1 = jnp.full((32, 128), E_, jnp.int32)
    for e in range(E_ - 1, -1, -1):
        le = jnp.where(e0 == e, neg, lg[e])
        e1 = jnp.where(le == m1, e, e1)

    ex = [jnp.exp(lg[e] - m0) for e in range(E_)]
    sm = ex[0]
    for e in range(1, E_):
        sm = sm + ex[e]
    inv_sm = 1.0 / sm
    p0 = inv_sm
    ex1 = jnp.zeros((32, 128), jnp.float32)
    for e in range(E_):
        ex1 = jnp.where(e1 == e, ex[e], ex1)
    p1 = ex1 * inv_sm

    tri_u = (lax.broadcasted_iota(jnp.int32, (128, 128), 0)
             <= lax.broadcasted_iota(jnp.int32, (128, 128), 1)
             ).astype(jnp.float32)
    strict_l32 = (lax.broadcasted_iota(jnp.int32, (32, 32), 1)
                  < lax.broadcasted_iota(jnp.int32, (32, 32), 0)
                  ).astype(jnp.float32)

    inv_sqrt2 = 0.7071067811865476
    rank0 = jnp.zeros((32, 128), jnp.float32)
    rank1 = jnp.zeros((32, 128), jnp.float32)
    for e in range(E_):
        m0e = (e0 == e).astype(jnp.float32)
        pref = jnp.dot(m0e, tri_u, preferred_element_type=jnp.float32) - m0e
        rsum = jnp.sum(m0e, axis=1, keepdims=True)
        roff = jnp.dot(strict_l32, rsum,
                       preferred_element_type=jnp.float32)
        rank0 = jnp.where(e0 == e, pref + roff, rank0)
        c0e = jnp.sum(rsum)

        m1e = (e1 == e).astype(jnp.float32)
        pref1 = jnp.dot(m1e, tri_u, preferred_element_type=jnp.float32) - m1e
        rsum1 = jnp.sum(m1e, axis=1, keepdims=True)
        roff1 = jnp.dot(strict_l32, rsum1,
                        preferred_element_type=jnp.float32)
        rank1 = jnp.where(e1 == e, pref1 + roff1 + c0e, rank1)

        sums_ref[0, e] = jnp.sum(ex[e] * inv_sm, axis=0)
        cdf = 0.5 * (1.0 + lax.erf((m1 - lg[e]) * (inv_sqrt2 / NOISE)))
        sums_ref[1, e] = jnp.sum(1.0 - cdf, axis=0)
        sums_ref[2, e] = jnp.sum(m0e + m1e, axis=0)

    rank0 = rank0.astype(jnp.int32)
    rank1 = rank1.astype(jnp.int32)
    sel0 = rank0 < CAP
    sel1 = rank1 < CAP
    slot0 = e0 * CAP + rank0
    slot1 = e1 * CAP + rank1

    scs_ref[0] = jnp.where(sel0, slot0, NSLOT)
    scs_ref[1] = jnp.where(sel1, slot1, NSLOT)
    cbs_ref[0] = jnp.where(sel0, slot0, 0)
    cbs_ref[1] = jnp.where(sel1, slot1, 0)
    w0_ref[...] = jnp.where(sel0, p0, 0.0)
    w1_ref[...] = jnp.where(sel1, p1, 0.0)


def _route(logits_t):
    return pl.pallas_call(
        _router_body,
        out_shape=(
            jax.ShapeDtypeStruct((K_, 32, 128), jnp.int32),
            jax.ShapeDtypeStruct((K_, 32, 128), jnp.int32),
            jax.ShapeDtypeStruct((32, 128), jnp.float32),
            jax.ShapeDtypeStruct((32, 128), jnp.float32),
            jax.ShapeDtypeStruct((3, E_, 128), jnp.float32),
        ),
    )(logits_t)


_DCH = 32
_DNC = ASG_W // _DCH


def _dispatch_rows(slots_hbm, x_hbm, xg_hbm, slots_v, buf0, buf1,
                   rs0, rs1, ss0, ss1):
    w = _wid()
    tokbase = lax.rem(w, 16) * ASG_W
    pltpu.sync_copy(slots_hbm.at[w], slots_v)

    bufs = (buf0, buf1)
    rsems = (rs0, rs1)
    ssems = (ss0, ss1)

    def read(c, b):
        return pltpu.async_copy(
            x_hbm.at[pl.ds(tokbase + c * _DCH, _DCH)], bufs[b], rsems[b])

    rd = [read(0, 0), None]
    sc = [None, None]
    for c in range(_DNC):
        b = c & 1
        rd[b].wait()
        if c + 1 < _DNC:
            nb = (c + 1) & 1
            if sc[nb] is not None:
                sc[nb].wait()
            rd[nb] = read(c + 1, nb)
        sc[b] = pltpu.async_copy(
            bufs[b], xg_hbm.at[slots_v.at[c]], ssems[b])
    for b in (0, 1):
        if sc[b] is not None:
            sc[b].wait()


_DW = D_ // 2


def _dispatch(slots, x_words):
    return pl.kernel(
        _dispatch_rows,
        out_type=jax.ShapeDtypeStruct((SZ, _DW), jnp.float32),
        mesh=_sc_mesh(),
        scratch_types=[
            pltpu.VMEM((_DNC, _DCH), jnp.int32),
            pltpu.VMEM((_DCH, _DW), jnp.float32),
            pltpu.VMEM((_DCH, _DW), jnp.float32),
            pltpu.SemaphoreType.DMA,
            pltpu.SemaphoreType.DMA,
            pltpu.SemaphoreType.DMA,
            pltpu.SemaphoreType.DMA,
        ],
    )(slots, x_words)


_HB = 1024
_NHC = H_ // _HB


def _mlp_body(xg_ref, w1_ref, b1_ref, w2_ref, b2_ref, out_ref):
    hc = pl.program_id(1)
    xb = xg_ref[...]
    w1 = w1_ref[0]
    h = lax.dot_general(xb, w1, (((1,), (1,)), ((), ())),
                        preferred_element_type=jnp.float32)
    h = h + b1_ref[0, 0]
    h = 0.5 * h * (1.0 + lax.erf(h * 0.7071067811865476))
    hb = h.astype(jnp.bfloat16)
    w2 = w2_ref[0]
    o = lax.dot_general(hb, w2, (((1,), (1,)), ((), ())),
                        preferred_element_type=jnp.float32)

    @pl.when(hc == 0)
    def _():
        out_ref[...] = o + b2_ref[0]

    @pl.when(hc != 0)
    def _():
        out_ref[...] += o


def _mlp(xg, fc1_w, fc1_b, fc2_w, fc2_b):
    return pl.pallas_call(
        _mlp_body,
        grid=(E_, _NHC),
        in_specs=[
            pl.BlockSpec((CAP, D_), lambda e, hc: (e, 0)),

            pl.BlockSpec((1, _HB, D_), lambda e, hc: (e, hc, 0)),
            pl.BlockSpec((1, 1, 1, _HB), lambda e, hc: (e, hc, 0, 0)),
            pl.BlockSpec((1, D_, _HB), lambda e, hc: (e, 0, hc)),
            pl.BlockSpec((1, 1, D_), lambda e, hc: (e, 0, 0)),
        ],
        out_specs=pl.BlockSpec((CAP, D_), lambda e, hc: (e, 0)),
        out_shape=jax.ShapeDtypeStruct((SZ, D_), jnp.float32),
    )(xg, fc1_w, fc1_b, fc2_w, fc2_b)


_TCH = 16
_NTC = TOK_W // _TCH


def _combine_rows(eb_hbm, cb0_hbm, cb1_hbm, w0_hbm, w1_hbm, out_hbm,
                  cb0_v, cb1_v, w0_v, w1_v,
                  bufa0, bufa1, bufb0, bufb1, obuf,
                  sa0, sa1, sb0, sb1):
    w = _wid()
    tb = w * TOK_W
    pltpu.sync_copy(cb0_hbm.at[pl.ds(tb, TOK_W)], cb0_v)
    pltpu.sync_copy(cb1_hbm.at[pl.ds(tb, TOK_W)], cb1_v)
    pltpu.sync_copy(w0_hbm.at[pl.ds(tb, TOK_W)], w0_v)
    pltpu.sync_copy(w1_hbm.at[pl.ds(tb, TOK_W)], w1_v)

    bufa = (bufa0, bufa1)
    bufb = (bufb0, bufb1)
    sas = (sa0, sa1)
    sbs = (sb0, sb1)

    def fetch(g, b):
        cpa = pltpu.async_copy(
            eb_hbm.at[cb0_v.at[pl.ds(g * _TCH, _TCH)]], bufa[b], sas[b])
        cpb = pltpu.async_copy(
            eb_hbm.at[cb1_v.at[pl.ds(g * _TCH, _TCH)]], bufb[b], sbs[b])
        return cpa, cpb

    pend = [fetch(0, 0), None]
    for g in range(_NTC):
        b = g & 1
        pend[b][0].wait()
        pend[b][1].wait()
        if g + 1 < _NTC:
            pend[(g + 1) & 1] = fetch(g + 1, (g + 1) & 1)

        def tok_body(t, _, b=b, g=g):
            w0 = w0_v[g * _TCH + t, :]
            w1 = w1_v[g * _TCH + t, :]

            def c_body(c, __):
                a = bufa[b][t, pl.ds(c * 16, 16)]
                bb = bufb[b][t, pl.ds(c * 16, 16)]
                r = (jnp.where(w0 > 0, w0 * a, 0.0)
                     + jnp.where(w1 > 0, w1 * bb, 0.0))
                obuf[t, pl.ds(c * 16, 16)] = r
                return __

            return lax.fori_loop(0, D_ // 16, c_body, _)

        lax.fori_loop(0, _TCH, tok_body, 0)
        pltpu.sync_copy(obuf, out_hbm.at[pl.ds(tb + g * _TCH, _TCH)])


def _combine(ebuf, cb0, cb1, w0x, w1x):
    return pl.kernel(
        _combine_rows,
        out_type=jax.ShapeDtypeStruct((N_, D_), jnp.float32),
        mesh=_sc_mesh(),
        scratch_types=[
            pltpu.VMEM((TOK_W,), jnp.int32),
            pltpu.VMEM((TOK_W,), jnp.int32),
            pltpu.VMEM((TOK_W, 16), jnp.float32),
            pltpu.VMEM((TOK_W, 16), jnp.float32),
            pltpu.VMEM((_TCH, D_), jnp.float32),
            pltpu.VMEM((_TCH, D_), jnp.float32),
            pltpu.VMEM((_TCH, D_), jnp.float32),
            pltpu.VMEM((_TCH, D_), jnp.float32),
            pltpu.VMEM((_TCH, D_), jnp.float32),
            pltpu.SemaphoreType.DMA,
            pltpu.SemaphoreType.DMA,
            pltpu.SemaphoreType.DMA,
            pltpu.SemaphoreType.DMA,
        ],
    )(ebuf, cb0, cb1, w0x, w1x)


def kernel(x, Wr, fc1_w, fc1_b, fc2_w, fc2_b):
    x_flat = x.reshape(N_, D_)
    logits = (x_flat @ Wr.T).astype(jnp.float32)

    scs, cbs, w0, w1, sums = _route(logits.T.reshape(E_, 32, 128))

    x_words = lax.bitcast_convert_type(
        x_flat.astype(jnp.bfloat16).reshape(N_, _DW, 2), jnp.float32)
    xg = _dispatch(scs.reshape(NW, _DNC, _DCH), x_words)
    xg_bf = lax.bitcast_convert_type(xg, jnp.bfloat16).reshape(SZ, D_)
    ebuf = _mlp(xg_bf, fc1_w.astype(jnp.bfloat16),
                fc1_b.reshape(E_, _NHC, 1, _HB),
                fc2_w.astype(jnp.bfloat16), fc2_b.reshape(E_, 1, D_))

    ones16 = jnp.ones((1, 16), jnp.float32)
    w0x = w0.reshape(N_, 1) * ones16
    w1x = w1.reshape(N_, 1) * ones16
    out_flat = _combine(ebuf, cbs[0].reshape(N_), cbs[1].reshape(N_),
                        w0x, w1x)

    importance = jnp.sum(sums[0], axis=1)
    load = jnp.sum(sums[1], axis=1)
    counts = jnp.sum(sums[2], axis=1)
    mi = importance.mean()
    li = jnp.var(importance) / (mi * mi + 1e-06)
    ml = load.mean()
    ll = jnp.var(load) / (ml * ml + 1e-06)
    aux_loss = 0.5 * (li + ll)
    dropped = jnp.sum(jnp.maximum(counts - CAP, 0.0))
    return (out_flat.reshape(B_, S_, D_), aux_loss, dropped, counts)


# speedup vs baseline: 1.6815x; 1.6815x over previous
import functools

import jax
import jax.numpy as jnp
from jax import lax
from jax.experimental import pallas as pl
from jax.experimental.pallas import tpu as pltpu
from jax.experimental.pallas import tpu_sc as plsc

B_, S_, D_ = 2, 2048, 1024
H_ = 4096
E_ = 8
K_ = 2
N_ = B_ * S_
CAP = int(round(K_ * N_ * 1.25 / E_))
NSLOT = E_ * CAP
SZ = NSLOT + 32
NOISE = 1.0 / E_
NC, NS = 2, 16
NW = NC * NS
ROWS_W = NSLOT // NW
TOK_W = N_ // NW
ASG_W = (N_ * K_) // NW


def _sc_mesh():
    return plsc.VectorSubcoreMesh(
        core_axis_name="c", subcore_axis_name="s",
        num_cores=NC, num_subcores=NS)


def _wid():
    return lax.axis_index("s") * NC + lax.axis_index("c")


def _router_body(lg_ref, scs_ref, cbs_ref, w0_ref, w1_ref, sums_ref):
    lg = [lg_ref[e] for e in range(E_)]
    neg = jnp.float32(-jnp.inf)

    m0 = lg[0]
    for e in range(1, E_):
        m0 = jnp.maximum(m0, lg[e])
    e0 = jnp.full((32, 128), E_, jnp.int32)
    for e in range(E_ - 1, -1, -1):
        e0 = jnp.where(lg[e] == m0, e, e0)

    m1 = jnp.full((32, 128), neg)
    for e in range(E_):
        le = jnp.where(e0 == e, neg, lg[e])
        m1 = jnp.maximum(m1, le)
    e1 = jnp.full((32, 128), E_, jnp.int32)
    for e in range(E_ - 1, -1, -1):
        le = jnp.where(e0 == e, neg, lg[e])
        e1 = jnp.where(le == m1, e, e1)

    ex = [jnp.exp(lg[e] - m0) for e in range(E_)]
    sm = ex[0]
    for e in range(1, E_):
        sm = sm + ex[e]
    inv_sm = 1.0 / sm
    p0 = inv_sm
    ex1 = jnp.zeros((32, 128), jnp.float32)
    for e in range(E_):
        ex1 = jnp.where(e1 == e, ex[e], ex1)
    p1 = ex1 * inv_sm

    tri_u = (lax.broadcasted_iota(jnp.int32, (128, 128), 0)
             <= lax.broadcasted_iota(jnp.int32, (128, 128), 1)
             ).astype(jnp.float32)
    strict_l32 = (lax.broadcasted_iota(jnp.int32, (32, 32), 1)
                  < lax.broadcasted_iota(jnp.int32, (32, 32), 0)
                  ).astype(jnp.float32)

    inv_sqrt2 = 0.7071067811865476
    rank0 = jnp.zeros((32, 128), jnp.float32)
    rank1 = jnp.zeros((32, 128), jnp.float32)
    for e in range(E_):
        m0e = (e0 == e).astype(jnp.float32)
        pref = jnp.dot(m0e, tri_u, preferred_element_type=jnp.float32) - m0e
        rsum = jnp.sum(m0e, axis=1, keepdims=True)
        roff = jnp.dot(strict_l32, rsum,
                       preferred_element_type=jnp.float32)
        rank0 = jnp.where(e0 == e, pref + roff, rank0)
        c0e = jnp.sum(rsum)

        m1e = (e1 == e).astype(jnp.float32)
        pref1 = jnp.dot(m1e, tri_u, preferred_element_type=jnp.float32) - m1e
        rsum1 = jnp.sum(m1e, axis=1, keepdims=True)
        roff1 = jnp.dot(strict_l32, rsum1,
                        preferred_element_type=jnp.float32)
        rank1 = jnp.where(e1 == e, pref1 + roff1 + c0e, rank1)

        sums_ref[0, e] = jnp.sum(ex[e] * inv_sm, axis=0)
        cdf = 0.5 * (1.0 + lax.erf((m1 - lg[e]) * (inv_sqrt2 / NOISE)))
        sums_ref[1, e] = jnp.sum(1.0 - cdf, axis=0)
        sums_ref[2, e] = jnp.sum(m0e + m1e, axis=0)

    rank0 = rank0.astype(jnp.int32)
    rank1 = rank1.astype(jnp.int32)
    sel0 = rank0 < CAP
    sel1 = rank1 < CAP
    slot0 = e0 * CAP + rank0
    slot1 = e1 * CAP + rank1

    scs_ref[0] = jnp.where(sel0, slot0, NSLOT)
    scs_ref[1] = jnp.where(sel1, slot1, NSLOT)
    cbs_ref[0] = jnp.where(sel0, slot0, 0)
    cbs_ref[1] = jnp.where(sel1, slot1, 0)
    w0_ref[...] = jnp.where(sel0, p0, 0.0)
    w1_ref[...] = jnp.where(sel1, p1, 0.0)


def _route(logits_t):
    return pl.pallas_call(
        _router_body,
        out_shape=(
            jax.ShapeDtypeStruct((K_, 32, 128), jnp.int32),
            jax.ShapeDtypeStruct((K_, 32, 128), jnp.int32),
            jax.ShapeDtypeStruct((32, 128), jnp.float32),
            jax.ShapeDtypeStruct((32, 128), jnp.float32),
            jax.ShapeDtypeStruct((3, E_, 128), jnp.float32),
        ),
    )(logits_t)


_DCH = 32
_DNC = ASG_W // _DCH


def _dispatch_rows(slots_hbm, x_hbm, xg_hbm, slots_v, buf0, buf1,
                   rs0, rs1, ss0, ss1):
    w = _wid()
    tokbase = lax.rem(w, 16) * ASG_W
    pltpu.sync_copy(slots_hbm.at[w], slots_v)

    bufs = (buf0, buf1)
    rsems = (rs0, rs1)
    ssems = (ss0, ss1)

    def read(c, b):
        return pltpu.async_copy(
            x_hbm.at[pl.ds(tokbase + c * _DCH, _DCH)], bufs[b], rsems[b])

    rd = [read(0, 0), None]
    sc = [None, None]
    for c in range(_DNC):
        b = c & 1
        rd[b].wait()
        if c + 1 < _DNC:
            nb = (c + 1) & 1
            if sc[nb] is not None:
                sc[nb].wait()
            rd[nb] = read(c + 1, nb)
        sc[b] = pltpu.async_copy(
            bufs[b], xg_hbm.at[slots_v.at[c]], ssems[b])
    for b in (0, 1):
        if sc[b] is not None:
            sc[b].wait()


def _dispatch(slots, x_flat):
    return pl.kernel(
        _dispatch_rows,
        out_type=jax.ShapeDtypeStruct((SZ, D_), jnp.float32),
        mesh=_sc_mesh(),
        scratch_types=[
            pltpu.VMEM((_DNC, _DCH), jnp.int32),
            pltpu.VMEM((_DCH, D_), jnp.float32),
            pltpu.VMEM((_DCH, D_), jnp.float32),
            pltpu.SemaphoreType.DMA,
            pltpu.SemaphoreType.DMA,
            pltpu.SemaphoreType.DMA,
            pltpu.SemaphoreType.DMA,
        ],
    )(slots, x_flat)


_HB = 1024
_NHC = H_ // _HB


def _mlp_body(xg_ref, w1_ref, b1_ref, w2_ref, b2_ref, out_ref):
    hc = pl.program_id(1)
    xb = xg_ref[...].astype(jnp.bfloat16)
    w1 = w1_ref[0]
    h = lax.dot_general(xb, w1, (((1,), (1,)), ((), ())),
                        preferred_element_type=jnp.float32)
    h = h + b1_ref[0, 0]
    h = 0.5 * h * (1.0 + lax.erf(h * 0.7071067811865476))
    hb = h.astype(jnp.bfloat16)
    w2 = w2_ref[0]
    o = lax.dot_general(hb, w2, (((1,), (1,)), ((), ())),
                        preferred_element_type=jnp.float32)

    @pl.when(hc == 0)
    def _():
        out_ref[...] = o + b2_ref[0]

    @pl.when(hc != 0)
    def _():
        out_ref[...] += o


def _mlp(xg, fc1_w, fc1_b, fc2_w, fc2_b):
    return pl.pallas_call(
        _mlp_body,
        grid=(E_, _NHC),
        in_specs=[
            pl.BlockSpec((CAP, D_), lambda e, hc: (e, 0)),

            pl.BlockSpec((1, _HB, D_), lambda e, hc: (e, hc, 0)),
            pl.BlockSpec((1, 1, 1, _HB), lambda e, hc: (e, hc, 0, 0)),
            pl.BlockSpec((1, D_, _HB), lambda e, hc: (e, 0, hc)),
            pl.BlockSpec((1, 1, D_), lambda e, hc: (e, 0, 0)),
        ],
        out_specs=pl.BlockSpec((CAP, D_), lambda e, hc: (e, 0)),
        out_shape=jax.ShapeDtypeStruct((SZ, D_), jnp.float32),
    )(xg, fc1_w, fc1_b, fc2_w, fc2_b)


_TCH = 16
_NTC = TOK_W // _TCH


def _combine_rows(eb_hbm, cb0_hbm, cb1_hbm, w0_hbm, w1_hbm, out_hbm,
                  cb0_v, cb1_v, w0_v, w1_v,
                  bufa0, bufa1, bufb0, bufb1, obuf,
                  sa0, sa1, sb0, sb1):
    w = _wid()
    tb = w * TOK_W
    pltpu.sync_copy(cb0_hbm.at[pl.ds(tb, TOK_W)], cb0_v)
    pltpu.sync_copy(cb1_hbm.at[pl.ds(tb, TOK_W)], cb1_v)
    pltpu.sync_copy(w0_hbm.at[pl.ds(tb, TOK_W)], w0_v)
    pltpu.sync_copy(w1_hbm.at[pl.ds(tb, TOK_W)], w1_v)

    bufa = (bufa0, bufa1)
    bufb = (bufb0, bufb1)
    sas = (sa0, sa1)
    sbs = (sb0, sb1)

    def fetch(g, b):
        cpa = pltpu.async_copy(
            eb_hbm.at[cb0_v.at[pl.ds(g * _TCH, _TCH)]], bufa[b], sas[b])
        cpb = pltpu.async_copy(
            eb_hbm.at[cb1_v.at[pl.ds(g * _TCH, _TCH)]], bufb[b], sbs[b])
        return cpa, cpb

    pend = [fetch(0, 0), None]
    for g in range(_NTC):
        b = g & 1
        pend[b][0].wait()
        pend[b][1].wait()
        if g + 1 < _NTC:
            pend[(g + 1) & 1] = fetch(g + 1, (g + 1) & 1)

        def tok_body(t, _, b=b, g=g):
            w0 = w0_v[g * _TCH + t, :]
            w1 = w1_v[g * _TCH + t, :]

            def c_body(c, __):
                a = bufa[b][t, pl.ds(c * 16, 16)]
                bb = bufb[b][t, pl.ds(c * 16, 16)]
                r = (jnp.where(w0 > 0, w0 * a, 0.0)
                     + jnp.where(w1 > 0, w1 * bb, 0.0))
                obuf[t, pl.ds(c * 16, 16)] = r
                return __

            return lax.fori_loop(0, D_ // 16, c_body, _)

        lax.fori_loop(0, _TCH, tok_body, 0)
        pltpu.sync_copy(obuf, out_hbm.at[pl.ds(tb + g * _TCH, _TCH)])


def _combine(ebuf, cb0, cb1, w0x, w1x):
    return pl.kernel(
        _combine_rows,
        out_type=jax.ShapeDtypeStruct((N_, D_), jnp.float32),
        mesh=_sc_mesh(),
        scratch_types=[
            pltpu.VMEM((TOK_W,), jnp.int32),
            pltpu.VMEM((TOK_W,), jnp.int32),
            pltpu.VMEM((TOK_W, 16), jnp.float32),
            pltpu.VMEM((TOK_W, 16), jnp.float32),
            pltpu.VMEM((_TCH, D_), jnp.float32),
            pltpu.VMEM((_TCH, D_), jnp.float32),
            pltpu.VMEM((_TCH, D_), jnp.float32),
            pltpu.VMEM((_TCH, D_), jnp.float32),
            pltpu.VMEM((_TCH, D_), jnp.float32),
            pltpu.SemaphoreType.DMA,
            pltpu.SemaphoreType.DMA,
            pltpu.SemaphoreType.DMA,
            pltpu.SemaphoreType.DMA,
        ],
    )(ebuf, cb0, cb1, w0x, w1x)


def kernel(x, Wr, fc1_w, fc1_b, fc2_w, fc2_b):
    x_flat = x.reshape(N_, D_)
    logits = (x_flat @ Wr.T).astype(jnp.float32)

    scs, cbs, w0, w1, sums = _route(logits.T.reshape(E_, 32, 128))

    xg = _dispatch(scs.reshape(NW, _DNC, _DCH), x_flat)
    ebuf = _mlp(xg, fc1_w.astype(jnp.bfloat16),
                fc1_b.reshape(E_, _NHC, 1, _HB),
                fc2_w.astype(jnp.bfloat16), fc2_b.reshape(E_, 1, D_))

    ones16 = jnp.ones((1, 16), jnp.float32)
    w0x = w0.reshape(N_, 1) * ones16
    w1x = w1.reshape(N_, 1) * ones16
    out_flat = _combine(ebuf, cbs[0].reshape(N_), cbs[1].reshape(N_),
                        w0x, w1x)

    importance = jnp.sum(sums[0], axis=1)
    load = jnp.sum(sums[1], axis=1)
    counts = jnp.sum(sums[2], axis=1)
    mi = importance.mean()
    li = jnp.var(importance) / (mi * mi + 1e-06)
    ml = load.mean()
    ll = jnp.var(load) / (ml * ml + 1e-06)
    aux_loss = 0.5 * (li + ll)
    dropped = jnp.sum(jnp.maximum(counts - CAP, 0.0))
    return (out_flat.reshape(B_, S_, D_), aux_loss, dropped, counts)


# speedup vs baseline: 1.7323x; 1.0302x over previous
import functools

import jax
import jax.numpy as jnp
from jax import lax
from jax.experimental import pallas as pl
from jax.experimental.pallas import tpu as pltpu
from jax.experimental.pallas import tpu_sc as plsc

B_, S_, D_ = 2, 2048, 1024
H_ = 4096
E_ = 8
K_ = 2
N_ = B_ * S_
CAP = int(round(K_ * N_ * 1.25 / E_))
NSLOT = E_ * CAP
SZ = NSLOT + 32
NOISE = 1.0 / E_
NC, NS = 2, 16
NW = NC * NS
ROWS_W = NSLOT // NW
TOK_W = N_ // NW
ASG_W = (N_ * K_) // NW


def _sc_mesh():
    return plsc.VectorSubcoreMesh(
        core_axis_name="c", subcore_axis_name="s",
        num_cores=NC, num_subcores=NS)


def _wid():
    return lax.axis_index("s") * NC + lax.axis_index("c")


def _router_body(lg_ref, scs_ref, cbs_ref, w0_ref, w1_ref, sums_ref):
    lg = [lg_ref[e] for e in range(E_)]
    neg = jnp.float32(-jnp.inf)

    m0 = lg[0]
    for e in range(1, E_):
        m0 = jnp.maximum(m0, lg[e])
    e0 = jnp.full((32, 128), E_, jnp.int32)
    for e in range(E_ - 1, -1, -1):
        e0 = jnp.where(lg[e] == m0, e, e0)

    m1 = jnp.full((32, 128), neg)
    for e in range(E_):
        le = jnp.where(e0 == e, neg, lg[e])
        m1 = jnp.maximum(m1, le)
    e1 = jnp.full((32, 128), E_, jnp.int32)
    for e in range(E_ - 1, -1, -1):
        le = jnp.where(e0 == e, neg, lg[e])
        e1 = jnp.where(le == m1, e, e1)

    ex = [jnp.exp(lg[e] - m0) for e in range(E_)]
    sm = ex[0]
    for e in range(1, E_):
        sm = sm + ex[e]
    inv_sm = 1.0 / sm
    p0 = inv_sm
    ex1 = jnp.zeros((32, 128), jnp.float32)
    for e in range(E_):
        ex1 = jnp.where(e1 == e, ex[e], ex1)
    p1 = ex1 * inv_sm

    tri_u = (lax.broadcasted_iota(jnp.int32, (128, 128), 0)
             <= lax.broadcasted_iota(jnp.int32, (128, 128), 1)
             ).astype(jnp.float32)
    strict_l32 = (lax.broadcasted_iota(jnp.int32, (32, 32), 1)
                  < lax.broadcasted_iota(jnp.int32, (32, 32), 0)
                  ).astype(jnp.float32)

    inv_sqrt2 = 0.7071067811865476
    rank0 = jnp.zeros((32, 128), jnp.float32)
    rank1 = jnp.zeros((32, 128), jnp.float32)
    for e in range(E_):
        m0e = (e0 == e).astype(jnp.float32)
        pref = jnp.dot(m0e, tri_u, preferred_element_type=jnp.float32) - m0e
        rsum = jnp.sum(m0e, axis=1, keepdims=True)
        roff = jnp.dot(strict_l32, rsum,
                       preferred_element_type=jnp.float32)
        rank0 = jnp.where(e0 == e, pref + roff, rank0)
        c0e = jnp.sum(rsum)

        m1e = (e1 == e).astype(jnp.float32)
        pref1 = jnp.dot(m1e, tri_u, preferred_element_type=jnp.float32) - m1e
        rsum1 = jnp.sum(m1e, axis=1, keepdims=True)
        roff1 = jnp.dot(strict_l32, rsum1,
                        preferred_element_type=jnp.float32)
        rank1 = jnp.where(e1 == e, pref1 + roff1 + c0e, rank1)

        sums_ref[0, e] = jnp.sum(ex[e] * inv_sm, axis=0)
        cdf = 0.5 * (1.0 + lax.erf((m1 - lg[e]) * (inv_sqrt2 / NOISE)))
        sums_ref[1, e] = jnp.sum(1.0 - cdf, axis=0)
        sums_ref[2, e] = jnp.sum(m0e + m1e, axis=0)

    rank0 = rank0.astype(jnp.int32)
    rank1 = rank1.astype(jnp.int32)
    sel0 = rank0 < CAP
    sel1 = rank1 < CAP
    slot0 = e0 * CAP + rank0
    slot1 = e1 * CAP + rank1

    scs_ref[0] = jnp.where(sel0, slot0, NSLOT)
    scs_ref[1] = jnp.where(sel1, slot1, NSLOT)
    cbs_ref[0] = jnp.where(sel0, slot0, 0)
    cbs_ref[1] = jnp.where(sel1, slot1, 0)
    w0_ref[...] = jnp.where(sel0, p0, 0.0)
    w1_ref[...] = jnp.where(sel1, p1, 0.0)


def _route(logits_t):
    return pl.pallas_call(
        _router_body,
        out_shape=(
            jax.ShapeDtypeStruct((K_, 32, 128), jnp.int32),
            jax.ShapeDtypeStruct((K_, 32, 128), jnp.int32),
            jax.ShapeDtypeStruct((32, 128), jnp.float32),
            jax.ShapeDtypeStruct((32, 128), jnp.float32),
            jax.ShapeDtypeStruct((3, E_, 128), jnp.float32),
        ),
    )(logits_t)


_DCH = 32
_DNC = ASG_W // _DCH


def _dispatch_rows(slots_hbm, x_hbm, xg_hbm, slots_v, buf0, buf1,
                   rs0, rs1, ss0, ss1):
    w = _wid()
    tokbase = lax.rem(w, 16) * ASG_W
    pltpu.sync_copy(slots_hbm.at[w], slots_v)

    bufs = (buf0, buf1)
    rsems = (rs0, rs1)
    ssems = (ss0, ss1)

    def read(c, b):
        return pltpu.async_copy(
            x_hbm.at[pl.ds(tokbase + c * _DCH, _DCH)], bufs[b], rsems[b])

    rd = [read(0, 0), None]
    sc = [None, None]
    for c in range(_DNC):
        b = c & 1
        rd[b].wait()
        if c + 1 < _DNC:
            nb = (c + 1) & 1
            if sc[nb] is not None:
                sc[nb].wait()
            rd[nb] = read(c + 1, nb)
        sc[b] = pltpu.async_copy(
            bufs[b], xg_hbm.at[slots_v.at[c]], ssems[b])
    for b in (0, 1):
        if sc[b] is not None:
            sc[b].wait()


def _dispatch(slots, x_flat):
    return pl.kernel(
        _dispatch_rows,
        out_type=jax.ShapeDtypeStruct((SZ, D_), jnp.float32),
        mesh=_sc_mesh(),
        scratch_types=[
            pltpu.VMEM((_DNC, _DCH), jnp.int32),
            pltpu.VMEM((_DCH, D_), jnp.float32),
            pltpu.VMEM((_DCH, D_), jnp.float32),
            pltpu.SemaphoreType.DMA,
            pltpu.SemaphoreType.DMA,
            pltpu.SemaphoreType.DMA,
            pltpu.SemaphoreType.DMA,
        ],
    )(slots, x_flat)


_HB = 2048
_NHC = H_ // _HB


def _mlp_body(xg_ref, w1_ref, b1_ref, w2_ref, b2_ref, out_ref):
    hc = pl.program_id(1)
    xb = xg_ref[...].astype(jnp.bfloat16)
    w1 = w1_ref[0]
    h = lax.dot_general(xb, w1, (((1,), (1,)), ((), ())),
                        preferred_element_type=jnp.float32)
    h = h + b1_ref[0, 0]
    h = 0.5 * h * (1.0 + lax.erf(h * 0.7071067811865476))
    hb = h.astype(jnp.bfloat16)
    w2 = w2_ref[0]
    o = lax.dot_general(hb, w2, (((1,), (1,)), ((), ())),
                        preferred_element_type=jnp.float32)

    @pl.when(hc == 0)
    def _():
        out_ref[...] = o + b2_ref[0]

    @pl.when(hc != 0)
    def _():
        out_ref[...] += o


def _mlp(xg, fc1_w, fc1_b, fc2_w, fc2_b):
    return pl.pallas_call(
        _mlp_body,
        grid=(E_, _NHC),
        in_specs=[
            pl.BlockSpec((CAP, D_), lambda e, hc: (e, 0)),

            pl.BlockSpec((1, _HB, D_), lambda e, hc: (e, hc, 0)),
            pl.BlockSpec((1, 1, 1, _HB), lambda e, hc: (e, hc, 0, 0)),
            pl.BlockSpec((1, D_, _HB), lambda e, hc: (e, 0, hc)),
            pl.BlockSpec((1, 1, D_), lambda e, hc: (e, 0, 0)),
        ],
        out_specs=pl.BlockSpec((CAP, D_), lambda e, hc: (e, 0)),
        out_shape=jax.ShapeDtypeStruct((SZ, D_), jnp.float32),
    )(xg, fc1_w, fc1_b, fc2_w, fc2_b)


_TCH = 16
_NTC = TOK_W // _TCH


def _combine_rows(eb_hbm, cb0_hbm, cb1_hbm, w0_hbm, w1_hbm, out_hbm,
                  cb0_v, cb1_v, w0_v, w1_v,
                  bufa0, bufa1, bufb0, bufb1, obuf,
                  sa0, sa1, sb0, sb1):
    w = _wid()
    tb = w * TOK_W
    pltpu.sync_copy(cb0_hbm.at[pl.ds(tb, TOK_W)], cb0_v)
    pltpu.sync_copy(cb1_hbm.at[pl.ds(tb, TOK_W)], cb1_v)
    pltpu.sync_copy(w0_hbm.at[pl.ds(tb, TOK_W)], w0_v)
    pltpu.sync_copy(w1_hbm.at[pl.ds(tb, TOK_W)], w1_v)

    bufa = (bufa0, bufa1)
    bufb = (bufb0, bufb1)
    sas = (sa0, sa1)
    sbs = (sb0, sb1)

    def fetch(g, b):
        cpa = pltpu.async_copy(
            eb_hbm.at[cb0_v.at[pl.ds(g * _TCH, _TCH)]], bufa[b], sas[b])
        cpb = pltpu.async_copy(
            eb_hbm.at[cb1_v.at[pl.ds(g * _TCH, _TCH)]], bufb[b], sbs[b])
        return cpa, cpb

    pend = [fetch(0, 0), None]
    for g in range(_NTC):
        b = g & 1
        pend[b][0].wait()
        pend[b][1].wait()
        if g + 1 < _NTC:
            pend[(g + 1) & 1] = fetch(g + 1, (g + 1) & 1)

        def tok_body(t, _, b=b, g=g):
            w0 = w0_v[g * _TCH + t, :]
            w1 = w1_v[g * _TCH + t, :]

            def c_body(c, __):
                a = bufa[b][t, pl.ds(c * 16, 16)]
                bb = bufb[b][t, pl.ds(c * 16, 16)]
                r = (jnp.where(w0 > 0, w0 * a, 0.0)
                     + jnp.where(w1 > 0, w1 * bb, 0.0))
                obuf[t, pl.ds(c * 16, 16)] = r
                return __

            return lax.fori_loop(0, D_ // 16, c_body, _)

        lax.fori_loop(0, _TCH, tok_body, 0)
        pltpu.sync_copy(obuf, out_hbm.at[pl.ds(tb + g * _TCH, _TCH)])


def _combine(ebuf, cb0, cb1, w0x, w1x):
    return pl.kernel(
        _combine_rows,
        out_type=jax.ShapeDtypeStruct((N_, D_), jnp.float32),
        mesh=_sc_mesh(),
        scratch_types=[
            pltpu.VMEM((TOK_W,), jnp.int32),
            pltpu.VMEM((TOK_W,), jnp.int32),
            pltpu.VMEM((TOK_W, 16), jnp.float32),
            pltpu.VMEM((TOK_W, 16), jnp.float32),
            pltpu.VMEM((_TCH, D_), jnp.float32),
            pltpu.VMEM((_TCH, D_), jnp.float32),
            pltpu.VMEM((_TCH, D_), jnp.float32),
            pltpu.VMEM((_TCH, D_), jnp.float32),
            pltpu.VMEM((_TCH, D_), jnp.float32),
            pltpu.SemaphoreType.DMA,
            pltpu.SemaphoreType.DMA,
            pltpu.SemaphoreType.DMA,
            pltpu.SemaphoreType.DMA,
        ],
    )(ebuf, cb0, cb1, w0x, w1x)


def kernel(x, Wr, fc1_w, fc1_b, fc2_w, fc2_b):
    x_flat = x.reshape(N_, D_)
    logits = (x_flat @ Wr.T).astype(jnp.float32)

    scs, cbs, w0, w1, sums = _route(logits.T.reshape(E_, 32, 128))

    xg = _dispatch(scs.reshape(NW, _DNC, _DCH), x_flat)
    ebuf = _mlp(xg, fc1_w.astype(jnp.bfloat16),
                fc1_b.reshape(E_, _NHC, 1, _HB),
                fc2_w.astype(jnp.bfloat16), fc2_b.reshape(E_, 1, D_))

    ones16 = jnp.ones((1, 16), jnp.float32)
    w0x = w0.reshape(N_, 1) * ones16
    w1x = w1.reshape(N_, 1) * ones16
    out_flat = _combine(ebuf, cbs[0].reshape(N_), cbs[1].reshape(N_),
                        w0x, w1x)

    importance = jnp.sum(sums[0], axis=1)
    load = jnp.sum(sums[1], axis=1)
    counts = jnp.sum(sums[2], axis=1)
    mi = importance.mean()
    li = jnp.var(importance) / (mi * mi + 1e-06)
    ml = load.mean()
    ll = jnp.var(load) / (ml * ml + 1e-06)
    aux_loss = 0.5 * (li + ll)
    dropped = jnp.sum(jnp.maximum(counts - CAP, 0.0))
    return (out_flat.reshape(B_, S_, D_), aux_loss, dropped, counts)


# speedup vs baseline: 1.7355x; 1.0018x over previous
import functools

import jax
import jax.numpy as jnp
from jax import lax
from jax.experimental import pallas as pl
from jax.experimental.pallas import tpu as pltpu
from jax.experimental.pallas import tpu_sc as plsc

B_, S_, D_ = 2, 2048, 1024
H_ = 4096
E_ = 8
K_ = 2
N_ = B_ * S_
CAP = int(round(K_ * N_ * 1.25 / E_))
NSLOT = E_ * CAP
SZ = NSLOT + 32
NOISE = 1.0 / E_
NC, NS = 2, 16
NW = NC * NS
TOK_W = N_ // NW
ASG_W = (N_ * K_) // NW


def _sc_mesh():
    return plsc.VectorSubcoreMesh(
        core_axis_name="c", subcore_axis_name="s",
        num_cores=NC, num_subcores=NS)


def _wid():
    return lax.axis_index("s") * NC + lax.axis_index("c")


def _router_body(lg_ref, scs_ref, cbs_ref, w0_ref, w1_ref, sums_ref):
    lg = [lg_ref[e] for e in range(E_)]
    neg = jnp.float32(-jnp.inf)

    m0 = lg[0]
    for e in range(1, E_):
        m0 = jnp.maximum(m0, lg[e])
    e0 = jnp.full((32, 128), E_, jnp.int32)
    for e in range(E_ - 1, -1, -1):
        e0 = jnp.where(lg[e] == m0, e, e0)

    m1 = jnp.full((32, 128), neg)
    for e in range(E_):
        le = jnp.where(e0 == e, neg, lg[e])
        m1 = jnp.maximum(m1, le)
    e1 = jnp.full((32, 128), E_, jnp.int32)
    for e in range(E_ - 1, -1, -1):
        le = jnp.where(e0 == e, neg, lg[e])
        e1 = jnp.where(le == m1, e, e1)

    ex = [jnp.exp(lg[e] - m0) for e in range(E_)]
    sm = ex[0]
    for e in range(1, E_):
        sm = sm + ex[e]
    inv_sm = 1.0 / sm
    p0 = inv_sm
    ex1 = jnp.zeros((32, 128), jnp.float32)
    for e in range(E_):
        ex1 = jnp.where(e1 == e, ex[e], ex1)
    p1 = ex1 * inv_sm

    tri_u = (lax.broadcasted_iota(jnp.int32, (128, 128), 0)
             <= lax.broadcasted_iota(jnp.int32, (128, 128), 1)
             ).astype(jnp.float32)
    strict_l32 = (lax.broadcasted_iota(jnp.int32, (32, 32), 1)
                  < lax.broadcasted_iota(jnp.int32, (32, 32), 0)
                  ).astype(jnp.float32)

    inv_sqrt2 = 0.7071067811865476
    rank0 = jnp.zeros((32, 128), jnp.float32)
    rank1 = jnp.zeros((32, 128), jnp.float32)
    for e in range(E_):
        m0e = (e0 == e).astype(jnp.float32)
        pref = jnp.dot(m0e, tri_u, preferred_element_type=jnp.float32) - m0e
        rsum = jnp.sum(m0e, axis=1, keepdims=True)
        roff = jnp.dot(strict_l32, rsum,
                       preferred_element_type=jnp.float32)
        rank0 = jnp.where(e0 == e, pref + roff, rank0)
        c0e = jnp.sum(rsum)

        m1e = (e1 == e).astype(jnp.float32)
        pref1 = jnp.dot(m1e, tri_u, preferred_element_type=jnp.float32) - m1e
        rsum1 = jnp.sum(m1e, axis=1, keepdims=True)
        roff1 = jnp.dot(strict_l32, rsum1,
                        preferred_element_type=jnp.float32)
        rank1 = jnp.where(e1 == e, pref1 + roff1 + c0e, rank1)

        sums_ref[0, e] = jnp.sum(ex[e] * inv_sm, axis=0)
        cdf = 0.5 * (1.0 + lax.erf((m1 - lg[e]) * (inv_sqrt2 / NOISE)))
        sums_ref[1, e] = jnp.sum(1.0 - cdf, axis=0)
        sums_ref[2, e] = jnp.sum(m0e + m1e, axis=0)

    rank0 = rank0.astype(jnp.int32)
    rank1 = rank1.astype(jnp.int32)
    sel0 = rank0 < CAP
    sel1 = rank1 < CAP
    slot0 = e0 * CAP + rank0
    slot1 = e1 * CAP + rank1

    scs_ref[0] = jnp.where(sel0, slot0, NSLOT)
    scs_ref[1] = jnp.where(sel1, slot1, NSLOT)
    cbs_ref[0] = jnp.where(sel0, slot0, 0)
    cbs_ref[1] = jnp.where(sel1, slot1, 0)
    w0_ref[...] = jnp.where(sel0, p0, 0.0)
    w1_ref[...] = jnp.where(sel1, p1, 0.0)


def _route(logits_t):
    return pl.pallas_call(
        _router_body,
        out_shape=(
            jax.ShapeDtypeStruct((K_, 32, 128), jnp.int32),
            jax.ShapeDtypeStruct((K_, 32, 128), jnp.int32),
            jax.ShapeDtypeStruct((32, 128), jnp.float32),
            jax.ShapeDtypeStruct((32, 128), jnp.float32),
            jax.ShapeDtypeStruct((3, E_, 128), jnp.float32),
        ),
    )(logits_t)


_DCH = 32
_DNC = ASG_W // _DCH


def _dispatch_rows(slots_hbm, x_hbm, xg_hbm, slots_v, buf0, buf1,
                   rs0, rs1, ss0, ss1):
    w = _wid()
    tokbase = lax.rem(w, 16) * ASG_W
    pltpu.sync_copy(slots_hbm.at[w], slots_v)

    bufs = (buf0, buf1)
    rsems = (rs0, rs1)
    ssems = (ss0, ss1)

    def read(c, b):
        return pltpu.async_copy(
            x_hbm.at[pl.ds(tokbase + c * _DCH, _DCH)], bufs[b], rsems[b])

    rd = [read(0, 0), None]
    sc = [None, None]
    for c in range(_DNC):
        b = c & 1
        rd[b].wait()
        if c + 1 < _DNC:
            nb = (c + 1) & 1
            if sc[nb] is not None:
                sc[nb].wait()
            rd[nb] = read(c + 1, nb)
        sc[b] = pltpu.async_copy(
            bufs[b], xg_hbm.at[slots_v.at[c]], ssems[b])
    for b in (0, 1):
        if sc[b] is not None:
            sc[b].wait()


def _dispatch(slots, x_flat):
    return pl.kernel(
        _dispatch_rows,
        out_type=jax.ShapeDtypeStruct((SZ, D_), jnp.float32),
        mesh=_sc_mesh(),
        scratch_types=[
            pltpu.VMEM((_DNC, _DCH), jnp.int32),
            pltpu.VMEM((_DCH, D_), jnp.float32),
            pltpu.VMEM((_DCH, D_), jnp.float32),
            pltpu.SemaphoreType.DMA,
            pltpu.SemaphoreType.DMA,
            pltpu.SemaphoreType.DMA,
            pltpu.SemaphoreType.DMA,
        ],
    )(slots, x_flat)


_HB = 2048
_NHC = H_ // _HB


def _mlp_body(xg_ref, w1_ref, b1_ref, w2_ref, b2_ref, out_ref):
    hc = pl.program_id(1)
    xb = xg_ref[...].astype(jnp.bfloat16)
    w1 = w1_ref[0]
    h = lax.dot_general(xb, w1, (((1,), (1,)), ((), ())),
                        preferred_element_type=jnp.float32)
    h = h + b1_ref[0, 0]
    h = 0.5 * h * (1.0 + lax.erf(h * 0.7071067811865476))
    hb = h.astype(jnp.bfloat16)
    w2 = w2_ref[0]
    o = lax.dot_general(hb, w2, (((1,), (1,)), ((), ())),
                        preferred_element_type=jnp.float32)

    @pl.when(hc == 0)
    def _():
        out_ref[...] = o + b2_ref[0]

    @pl.when(hc != 0)
    def _():
        out_ref[...] += o


def _mlp(xg, fc1_w, fc1_b, fc2_w, fc2_b):
    return pl.pallas_call(
        _mlp_body,
        grid=(E_, _NHC),
        in_specs=[
            pl.BlockSpec((CAP, D_), lambda e, hc: (e, 0)),

            pl.BlockSpec((1, _HB, D_), lambda e, hc: (e, hc, 0)),
            pl.BlockSpec((1, 1, 1, _HB), lambda e, hc: (e, hc, 0, 0)),
            pl.BlockSpec((1, D_, _HB), lambda e, hc: (e, 0, hc)),
            pl.BlockSpec((1, 1, D_), lambda e, hc: (e, 0, 0)),
        ],
        out_specs=pl.BlockSpec((CAP, D_), lambda e, hc: (e, 0)),
        out_shape=jax.ShapeDtypeStruct((SZ, D_), jnp.float32),
    )(xg, fc1_w, fc1_b, fc2_w, fc2_b)


_TCH = 16
_NTC = TOK_W // _TCH


def _combine_rows(eb_hbm, cb0_hbm, cb1_hbm, w0_hbm, w1_hbm, out_hbm,
                  cb0_v, cb1_v, w0_v, w1_v,
                  bufa0, bufa1, bufb0, bufb1, obuf,
                  sa0, sa1, sb0, sb1):
    w = _wid()
    tb = w * TOK_W
    pltpu.sync_copy(cb0_hbm.at[pl.ds(tb, TOK_W)], cb0_v)
    pltpu.sync_copy(cb1_hbm.at[pl.ds(tb, TOK_W)], cb1_v)
    pltpu.sync_copy(w0_hbm.at[pl.ds(tb, TOK_W)], w0_v)
    pltpu.sync_copy(w1_hbm.at[pl.ds(tb, TOK_W)], w1_v)

    bufa = (bufa0, bufa1)
    bufb = (bufb0, bufb1)
    sas = (sa0, sa1)
    sbs = (sb0, sb1)

    def fetch(g, b):
        cpa = pltpu.async_copy(
            eb_hbm.at[cb0_v.at[pl.ds(g * _TCH, _TCH)]], bufa[b], sas[b])
        cpb = pltpu.async_copy(
            eb_hbm.at[cb1_v.at[pl.ds(g * _TCH, _TCH)]], bufb[b], sbs[b])
        return cpa, cpb

    pend = [fetch(0, 0), None]
    for g in range(_NTC):
        b = g & 1
        pend[b][0].wait()
        pend[b][1].wait()
        if g + 1 < _NTC:
            pend[(g + 1) & 1] = fetch(g + 1, (g + 1) & 1)

        def tok_body(t, _, b=b, g=g):
            w0 = w0_v[g * _TCH + t, :]
            w1 = w1_v[g * _TCH + t, :]

            def c_body(c, __):
                a = bufa[b][t, pl.ds(c * 16, 16)]
                bb = bufb[b][t, pl.ds(c * 16, 16)]
                r = (jnp.where(w0 > 0, w0 * a, 0.0)
                     + jnp.where(w1 > 0, w1 * bb, 0.0))
                obuf[t, pl.ds(c * 16, 16)] = r
                return __

            return lax.fori_loop(0, D_ // 16, c_body, _)

        lax.fori_loop(0, _TCH, tok_body, 0)
        pltpu.sync_copy(obuf, out_hbm.at[pl.ds(tb + g * _TCH, _TCH)])


def _combine(ebuf, cb0, cb1, w0x, w1x):
    return pl.kernel(
        _combine_rows,
        out_type=jax.ShapeDtypeStruct((N_, D_), jnp.float32),
        mesh=_sc_mesh(),
        scratch_types=[
            pltpu.VMEM((TOK_W,), jnp.int32),
            pltpu.VMEM((TOK_W,), jnp.int32),
            pltpu.VMEM((TOK_W, 16), jnp.float32),
            pltpu.VMEM((TOK_W, 16), jnp.float32),
            pltpu.VMEM((_TCH, D_), jnp.float32),
            pltpu.VMEM((_TCH, D_), jnp.float32),
            pltpu.VMEM((_TCH, D_), jnp.float32),
            pltpu.VMEM((_TCH, D_), jnp.float32),
            pltpu.VMEM((_TCH, D_), jnp.float32),
            pltpu.SemaphoreType.DMA,
            pltpu.SemaphoreType.DMA,
            pltpu.SemaphoreType.DMA,
            pltpu.SemaphoreType.DMA,
        ],
    )(ebuf, cb0, cb1, w0x, w1x)


def kernel(x, Wr, fc1_w, fc1_b, fc2_w, fc2_b):
    x_flat = x.reshape(N_, D_)
    logits = (x_flat @ Wr.T).astype(jnp.float32)

    scs, cbs, w0, w1, sums = _route(logits.T.reshape(E_, 32, 128))

    xg = _dispatch(scs.reshape(NW, _DNC, _DCH), x_flat)
    ebuf = _mlp(xg, fc1_w.astype(jnp.bfloat16),
                fc1_b.reshape(E_, _NHC, 1, _HB),
                fc2_w.astype(jnp.bfloat16), fc2_b.reshape(E_, 1, D_))

    ones16 = jnp.ones((1, 16), jnp.float32)
    w0x = w0.reshape(N_, 1) * ones16
    w1x = w1.reshape(N_, 1) * ones16
    out_flat = _combine(ebuf, cbs[0].reshape(N_), cbs[1].reshape(N_),
                        w0x, w1x)

    importance = jnp.sum(sums[0], axis=1)
    load = jnp.sum(sums[1], axis=1)
    counts = jnp.sum(sums[2], axis=1)
    mi = importance.mean()
    li = jnp.var(importance) / (mi * mi + 1e-06)
    ml = load.mean()
    ll = jnp.var(load) / (ml * ml + 1e-06)
    aux_loss = 0.5 * (li + ll)
    dropped = jnp.sum(jnp.maximum(counts - CAP, 0.0))
    return (out_flat.reshape(B_, S_, D_), aux_loss, dropped, counts)
